# Initial kernel scaffold; baseline (speedup 1.0000x reference)
import jax
import jax.numpy as jnp
from jax.experimental import pallas as pl


def kernel(hidden_phoneme_sequence, sequence_mask, frame_masks, pitch_target, energy_target, duration_target, duration_scale, pitch_scale, energy_scale, dur_c1w, dur_c1b, dur_ln1s, dur_ln1b, dur_c2w, dur_c2b, dur_ln2s, dur_ln2b, dur_lw, dur_lb, pit_c1w, pit_c1b, pit_ln1s, pit_ln1b, pit_c2w, pit_c2b, pit_ln2s, pit_ln2b, pit_lw, pit_lb, ene_c1w, ene_c1b, ene_ln1s, ene_ln1b, ene_c2w, ene_c2b, ene_ln2s, ene_ln2b, ene_lw, ene_lb, pitch_bins, energy_bins, pitch_emb, energy_emb):
    raise NotImplementedError("write your pallas kernel here")


# speedup vs baseline: 21.2580x; 21.2580x over previous
import functools

import jax
import jax.numpy as jnp
from jax.experimental import pallas as pl
from jax.experimental.pallas import tpu as pltpu

B, L, M, E = 16, 512, 2048, 256
F, K, NB = 256, 3, 256
_F32 = jnp.float32
_I32 = jnp.int32


def _shift_dn(x):
    return jnp.concatenate([jnp.zeros((1, x.shape[1]), x.dtype), x[:-1]], axis=0)


def _shift_up(x):
    return jnp.concatenate([x[1:], jnp.zeros((1, x.shape[1]), x.dtype)], axis=0)


def _layer_norm(h, s, b):
    mu = jnp.mean(h, axis=-1, keepdims=True)
    var = jnp.mean((h - mu) * (h - mu), axis=-1, keepdims=True)
    return (h - mu) / jnp.sqrt(var + 1e-5) * s[None, :] + b[None, :]


def _conv3(x, w, bias):
    h = jnp.dot(x, w[1], preferred_element_type=_F32)
    h = h + jnp.dot(_shift_dn(x), w[0], preferred_element_type=_F32)
    h = h + jnp.dot(_shift_up(x), w[2], preferred_element_type=_F32)
    return h + bias[None, :]


def _predictor(x, c1w, c1b, ln1s, ln1b, c2w, c2b, ln2s, ln2b, lw, lb):
    h = jax.nn.relu(_conv3(x, c1w, c1b))
    h = _layer_norm(h, ln1s, ln1b)
    h = jax.nn.relu(_conv3(h, c2w, c2b))
    h = _layer_norm(h, ln2s, ln2b)
    return jnp.sum(h * lw[None, :], axis=1) + lb


def _bucket_emb(target, bins, emb):
    idx = jnp.minimum(
        jnp.sum((bins[None, :] < target[:, None]).astype(_I32), axis=1), NB - 1)
    oh = (idx[:, None] == jax.lax.broadcasted_iota(_I32, (L, NB), 1)).astype(_F32)
    return jnp.dot(oh, emb, preferred_element_type=_F32)


def _tc_body(x_ref, pt_ref, et_ref, dur_ref,
             dw1, db1, ds1, dbb1, dw2, db2, ds2, dbb2, dlw, dlb,
             pw1, pb1, ps1, pbb1, pw2, pb2, ps2, pbb2, plw, plb,
             ew1, eb1, es1, ebb1, ew2, eb2, es2, ebb2, elw, elb,
             pbins, ebins, pemb, eemb,
             logd_ref, pitch_ref, energy_ref, xout_ref):
    x0 = x_ref[0]
    ptv = pt_ref[0, 0, :]
    etv = et_ref[0, 0, :]

    p_emb = _bucket_emb(ptv, pbins[0], pemb[...])
    e_emb = _bucket_emb(etv, ebins[0], eemb[...])
    x1 = x0 + p_emb
    x2 = x1 + e_emb

    logd_ref[0, 0, :] = _predictor(
        x0, dw1[...], db1[0], ds1[0], dbb1[0], dw2[...], db2[0], ds2[0],
        dbb2[0], dlw[0], dlb[0, 0])
    pitch_ref[0, 0, :] = _predictor(
        x0, pw1[...], pb1[0], ps1[0], pbb1[0], pw2[...], pb2[0], ps2[0],
        pbb2[0], plw[0], plb[0, 0])
    energy_ref[0, 0, :] = _predictor(
        x1, ew1[...], eb1[0], es1[0], ebb1[0], ew2[...], eb2[0], es2[0],
        ebb2[0], elw[0], elb[0, 0])

    df = dur_ref[0, 0, :].astype(_F32)
    tri = (jax.lax.broadcasted_iota(_I32, (L, L), 0)
           < jax.lax.broadcasted_iota(_I32, (L, L), 1)).astype(_F32)
    excl = jnp.dot(df[None, :], tri, preferred_element_type=_F32)[0].astype(_I32)
    miota = jax.lax.broadcasted_iota(_I32, (M, 1), 0)
    cnt = jnp.sum((excl[None, :] <= miota).astype(_I32), axis=1)
    gidx = cnt - 1
    oh = (gidx[:, None] == jax.lax.broadcasted_iota(_I32, (M, L), 1)).astype(_F32)
    xout_ref[0] = jnp.dot(oh, x2, preferred_element_type=_F32)


def _row3(shape):
    return pl.BlockSpec((1, 1, shape), lambda b: (b, 0, 0))


def _const(*shape):
    nd = len(shape)
    return pl.BlockSpec(shape, lambda b, _n=nd: (0,) * _n)


@jax.jit
def _run(x0, pt, et, dur, wts, pbins, ebins, pemb, eemb):
    w_specs = []
    for _ in range(3):
        w_specs += [
            _const(K, E, F), _const(1, F), _const(1, F), _const(1, F),
            _const(K, F, F), _const(1, F), _const(1, F), _const(1, F),
            _const(1, F), _const(1, 1),
        ]
    out_shapes = (
        jax.ShapeDtypeStruct((B, 1, L), _F32),
        jax.ShapeDtypeStruct((B, 1, L), _F32),
        jax.ShapeDtypeStruct((B, 1, L), _F32),
        jax.ShapeDtypeStruct((B, M, E), _F32),
    )
    out_specs = (_row3(L), _row3(L), _row3(L),
                 pl.BlockSpec((1, M, E), lambda b: (b, 0, 0)))
    logd, pitch, energy, xout = pl.pallas_call(
        _tc_body,
        grid=(B,),
        in_specs=[
            pl.BlockSpec((1, L, E), lambda b: (b, 0, 0)),
            _row3(L), _row3(L), _row3(L),
            *w_specs,
            _const(1, NB), _const(1, NB), _const(NB, E), _const(NB, E),
        ],
        out_specs=out_specs,
        out_shape=out_shapes,
    )(x0, pt, et, dur, *wts, pbins, ebins, pemb, eemb)
    return logd, pitch, energy, xout


def kernel(hidden_phoneme_sequence, sequence_mask, frame_masks, pitch_target,
           energy_target, duration_target, duration_scale, pitch_scale,
           energy_scale,
           dur_c1w, dur_c1b, dur_ln1s, dur_ln1b, dur_c2w, dur_c2b,
           dur_ln2s, dur_ln2b, dur_lw, dur_lb,
           pit_c1w, pit_c1b, pit_ln1s, pit_ln1b, pit_c2w, pit_c2b,
           pit_ln2s, pit_ln2b, pit_lw, pit_lb,
           ene_c1w, ene_c1b, ene_ln1s, ene_ln1b, ene_c2w, ene_c2b,
           ene_ln2s, ene_ln2b, ene_lw, ene_lb,
           pitch_bins, energy_bins, pitch_emb, energy_emb):
    r2 = lambda a: a.reshape(1, -1)
    wts = []
    for t in ((dur_c1w, dur_c1b, dur_ln1s, dur_ln1b, dur_c2w, dur_c2b,
               dur_ln2s, dur_ln2b, dur_lw, dur_lb),
              (pit_c1w, pit_c1b, pit_ln1s, pit_ln1b, pit_c2w, pit_c2b,
               pit_ln2s, pit_ln2b, pit_lw, pit_lb),
              (ene_c1w, ene_c1b, ene_ln1s, ene_ln1b, ene_c2w, ene_c2b,
               ene_ln2s, ene_ln2b, ene_lw, ene_lb)):
        c1w, c1b, ln1s, ln1b, c2w, c2b, ln2s, ln2b, lw, lb = t
        wts += [c1w, r2(c1b), r2(ln1s), r2(ln1b), c2w, r2(c2b), r2(ln2s),
                r2(ln2b), lw.reshape(1, F), lb.reshape(1, 1)]

    logd, pitch, energy, xout = _run(
        hidden_phoneme_sequence,
        pitch_target.reshape(B, 1, L), energy_target.reshape(B, 1, L),
        duration_target.reshape(B, 1, L).astype(_I32),
        tuple(wts), r2(pitch_bins), r2(energy_bins), pitch_emb, energy_emb)
    return (logd.reshape(B, L), pitch.reshape(B, L), energy.reshape(B, L),
            xout, frame_masks)
